# Initial kernel scaffold; baseline (speedup 1.0000x reference)
import jax
import jax.numpy as jnp
from jax.experimental import pallas as pl


def kernel(x, edge_index, w_mul, W, b):
    raise NotImplementedError("write your pallas kernel here")


# speedup vs baseline: 17.9322x; 17.9322x over previous
import functools

import jax
import jax.numpy as jnp
from jax import lax
from jax.experimental import pallas as pl
from jax.experimental.pallas import tpu as pltpu
from jax.experimental.pallas import tpu_sc as plsc

N = 10000
NP = 10240
D_IN = 128
D_PAD = 16

NC = 2
NS = 16
NW = NC * NS
BLK = 128


def _mm_body(x_ref, w_ref, o_ref):
    o_ref[...] = jnp.dot(x_ref[...], w_ref[...],
                         preferred_element_type=jnp.float32)


def _matmul(x, w_pad):
    return pl.pallas_call(
        _mm_body,
        out_shape=jax.ShapeDtypeStruct((NP, D_PAD), jnp.float32),
    )(x, w_pad)


def _sc_body(h_hbm, src_hbm, dst_hbm, w_hbm, out_hbm,
             src_v, dst_v, w_v, gbuf, h_sh, acc_sh, bpt):
    c = lax.axis_index("c")
    s = lax.axis_index("s")
    wid = c * NS + s

    zeros16 = jnp.zeros((16,), jnp.float32)

    pltpu.sync_copy(src_hbm.at[wid], src_v)
    pltpu.sync_copy(dst_hbm.at[wid], dst_v)
    pltpu.sync_copy(w_hbm.at[wid], w_v)

    def _z(i, _):
        gbuf[i, :] = zeros16
        return 0
    lax.fori_loop(0, BLK, _z, 0)

    rows = NP // NS
    base = s * rows

    def _zacc(k, _):
        pltpu.sync_copy(gbuf, acc_sh.at[pl.ds(base + k * BLK, BLK)])
        return 0
    lax.fori_loop(0, rows // BLK, _zacc, 0)
    pltpu.sync_copy(h_hbm.at[pl.ds(base, rows)], h_sh.at[pl.ds(base, rows)])
    plsc.subcore_barrier()

    def _blk(jb, _):
        pltpu.sync_copy(h_sh.at[src_v.at[jb]], gbuf)
        wbase = jb * BLK
        for g in range(BLK // 16):
            wvec = w_v[pl.ds(wbase + g * 16, 16)]
            for k in range(16):
                wb = jnp.take_along_axis(
                    wvec, jnp.full((16,), k, jnp.int32), axis=0)
                o = g * 16 + k
                gbuf[o, :] = gbuf[o, :] * wb
        pltpu.sync_copy(gbuf, acc_sh.at[dst_v.at[jb]], add=True)
        return 0
    lax.fori_loop(0, bpt, _blk, 0)

    plsc.subcore_barrier()
    pltpu.sync_copy(acc_sh.at[pl.ds(base, rows)],
                    out_hbm.at[c, pl.ds(base, rows)])


def _edge_aggregate(h, src, dst, w, bpt):
    mesh = plsc.VectorSubcoreMesh(core_axis_name="c", subcore_axis_name="s")
    body = functools.partial(_sc_body, bpt=bpt)
    return pl.kernel(
        body,
        out_type=jax.ShapeDtypeStruct((NC, NP, D_PAD), jnp.float32),
        mesh=mesh,
        compiler_params=pltpu.CompilerParams(use_tc_tiling_on_sc=False),
        scratch_types=[
            pltpu.VMEM((bpt, BLK), jnp.int32),
            pltpu.VMEM((bpt, BLK), jnp.int32),
            pltpu.VMEM((bpt * BLK,), jnp.float32),
            pltpu.VMEM((BLK, D_PAD), jnp.float32),
            pltpu.VMEM_SHARED((NP, D_PAD), jnp.float32),
            pltpu.VMEM_SHARED((NP, D_PAD), jnp.float32),
        ],
    )(h, src, dst, w)


@jax.jit
def kernel(x, edge_index, w_mul, W, b):
    e = edge_index.shape[1]
    bpt = -(-e // (NW * BLK))
    ep = NW * bpt * BLK
    pad = ep - e

    src = jnp.concatenate(
        [edge_index[0].astype(jnp.int32), jnp.zeros((pad,), jnp.int32)]
    ).reshape(NW, bpt, BLK)
    dst = jnp.concatenate(
        [edge_index[1].astype(jnp.int32), jnp.zeros((pad,), jnp.int32)]
    ).reshape(NW, bpt, BLK)
    w = jnp.concatenate(
        [w_mul, jnp.zeros((pad,), jnp.float32)]
    ).reshape(NW, bpt * BLK)

    w_pad = jnp.zeros((D_IN, D_PAD), jnp.float32).at[:, : W.shape[1]].set(W)
    x_pad = jnp.zeros((NP, D_IN), jnp.float32).at[:N].set(x)
    h = _matmul(x_pad, w_pad)

    partials = _edge_aggregate(h, src, dst, w, bpt)
    out = partials[0] + partials[1]
    return out[:N, : W.shape[1]] + b


# speedup vs baseline: 20.9621x; 1.1690x over previous
import functools

import jax
import jax.numpy as jnp
from jax import lax
from jax.experimental import pallas as pl
from jax.experimental.pallas import tpu as pltpu
from jax.experimental.pallas import tpu_sc as plsc

N = 10000
NP = 10240
D_IN = 128
D_PAD = 16

NC = 2
NS = 16
NW = NC * NS
BLK = 128


def _mm_body(x_ref, w_ref, o_ref):
    o_ref[...] = jnp.dot(x_ref[...], w_ref[...],
                         preferred_element_type=jnp.float32)


def _matmul(x, w_pad):
    return pl.pallas_call(
        _mm_body,
        out_shape=jax.ShapeDtypeStruct((NP, D_PAD), jnp.float32),
    )(x, w_pad)


def _sc_body(h_hbm, src_hbm, dst_hbm, w_hbm, out_hbm,
             b0, b1, b2, b3, src_v, dst_v, w_v, h_sh, acc_sh,
             gs0, gs1, gs2, gs3, ss0, ss1, ss2, ss3, bpt):
    c = lax.axis_index("c")
    s = lax.axis_index("s")
    wid = c * NS + s
    bufs = (b0, b1, b2, b3)
    gsems = (gs0, gs1, gs2, gs3)
    ssems = (ss0, ss1, ss2, ss3)

    zeros16 = jnp.zeros((16,), jnp.float32)

    pltpu.sync_copy(src_hbm.at[wid], src_v)
    pltpu.sync_copy(dst_hbm.at[wid], dst_v)
    pltpu.sync_copy(w_hbm.at[wid], w_v)

    def _z(i, _):
        b0[i, :] = zeros16
        return 0
    lax.fori_loop(0, BLK, _z, 0)

    rows = NP // NS
    base = s * rows

    def _zacc(k, _):
        pltpu.sync_copy(b0, acc_sh.at[pl.ds(base + k * BLK, BLK)])
        return 0
    lax.fori_loop(0, rows // BLK, _zacc, 0)
    pltpu.sync_copy(h_hbm.at[pl.ds(base, rows)], h_sh.at[pl.ds(base, rows)])
    plsc.subcore_barrier()

    def start_gather(jb, buf, sem):
        pltpu.async_copy(h_sh.at[src_v.at[jb]], buf, sem)

    def wait_gather(jb, buf, sem):
        pltpu.make_async_copy(h_sh.at[src_v.at[jb]], buf, sem).wait()

    def start_scatter(jb, buf, sem):
        pltpu.async_copy(buf, acc_sh.at[dst_v.at[jb]], sem, add=True)

    def wait_scatter(jb, buf, sem):
        pltpu.make_async_copy(buf, acc_sh.at[dst_v.at[jb]], sem).wait()

    def multiply(buf, jb):
        wbase = jb * BLK
        for g in range(BLK // 16):
            wvec = w_v[pl.ds(wbase + g * 16, 16)]
            for k in range(16):
                wb = jnp.take_along_axis(
                    wvec, jnp.full((16,), k, jnp.int32), axis=0)
                o = g * 16 + k
                buf[o, :] = buf[o, :] * wb

    start_gather(0, b0, gs0)
    start_gather(1, b1, gs1)

    def _quad(j4, _):
        for r in range(4):
            jb = j4 * 4 + r
            s2 = (r + 2) % 4
            wait_gather(jb, bufs[r], gsems[r])
            multiply(bufs[r], jb)
            start_scatter(jb, bufs[r], ssems[r])

            @pl.when(jb + 2 < bpt)
            def _():
                @pl.when(jb - 2 >= 0)
                def _():
                    wait_scatter(jb - 2, bufs[s2], ssems[s2])
                start_gather(jb + 2, bufs[s2], gsems[s2])
        return 0
    lax.fori_loop(0, bpt // 4, _quad, 0)

    for r in range(4):
        wait_scatter(bpt - 4 + r, bufs[r], ssems[r])

    plsc.subcore_barrier()
    pltpu.sync_copy(acc_sh.at[pl.ds(base, rows)],
                    out_hbm.at[c, pl.ds(base, rows)])


def _edge_aggregate(h, src, dst, w, bpt):
    mesh = plsc.VectorSubcoreMesh(core_axis_name="c", subcore_axis_name="s")
    body = functools.partial(_sc_body, bpt=bpt)
    return pl.kernel(
        body,
        out_type=jax.ShapeDtypeStruct((NC, NP, D_PAD), jnp.float32),
        mesh=mesh,
        compiler_params=pltpu.CompilerParams(use_tc_tiling_on_sc=False),
        scratch_types=[
            pltpu.VMEM((BLK, D_PAD), jnp.float32),
            pltpu.VMEM((BLK, D_PAD), jnp.float32),
            pltpu.VMEM((BLK, D_PAD), jnp.float32),
            pltpu.VMEM((BLK, D_PAD), jnp.float32),
            pltpu.VMEM((bpt, BLK), jnp.int32),
            pltpu.VMEM((bpt, BLK), jnp.int32),
            pltpu.VMEM((bpt * BLK,), jnp.float32),
            pltpu.VMEM_SHARED((NP, D_PAD), jnp.float32),
            pltpu.VMEM_SHARED((NP, D_PAD), jnp.float32),
            pltpu.SemaphoreType.DMA,
            pltpu.SemaphoreType.DMA,
            pltpu.SemaphoreType.DMA,
            pltpu.SemaphoreType.DMA,
            pltpu.SemaphoreType.DMA,
            pltpu.SemaphoreType.DMA,
            pltpu.SemaphoreType.DMA,
            pltpu.SemaphoreType.DMA,
        ],
    )(h, src, dst, w)


@jax.jit
def kernel(x, edge_index, w_mul, W, b):
    e = edge_index.shape[1]
    bpt = -(-e // (NW * BLK))
    bpt += (-bpt) % 4
    ep = NW * bpt * BLK
    pad = ep - e

    src = jnp.concatenate(
        [edge_index[0].astype(jnp.int32), jnp.zeros((pad,), jnp.int32)]
    ).reshape(NW, bpt, BLK)
    dst = jnp.concatenate(
        [edge_index[1].astype(jnp.int32), jnp.zeros((pad,), jnp.int32)]
    ).reshape(NW, bpt, BLK)
    w = jnp.concatenate(
        [w_mul, jnp.zeros((pad,), jnp.float32)]
    ).reshape(NW, bpt * BLK)

    w_pad = jnp.zeros((D_IN, D_PAD), jnp.float32).at[:, : W.shape[1]].set(W)
    x_pad = jnp.zeros((NP, D_IN), jnp.float32).at[:N].set(x)
    h = _matmul(x_pad, w_pad)

    partials = _edge_aggregate(h, src, dst, w, bpt)
    out = partials[0] + partials[1]
    return out[:N, : W.shape[1]] + b


# speedup vs baseline: 24.2504x; 1.1569x over previous
import functools

import jax
import jax.numpy as jnp
from jax import lax
from jax.experimental import pallas as pl
from jax.experimental.pallas import tpu as pltpu
from jax.experimental.pallas import tpu_sc as plsc

N = 10000
NP = 10240
D_IN = 128
D_PAD = 16

NC = 2
NS = 16
NW = NC * NS
BLK = 128


def _mm_body(x_ref, w_ref, o_ref):
    o_ref[...] = jnp.dot(x_ref[...], w_ref[...],
                         preferred_element_type=jnp.float32)


_MM_BLK = 1280


def _matmul(x, w_pad):
    return pl.pallas_call(
        _mm_body,
        grid=(NP // _MM_BLK,),
        in_specs=[
            pl.BlockSpec((_MM_BLK, D_IN), lambda i: (i, 0)),
            pl.BlockSpec((D_IN, D_PAD), lambda i: (0, 0)),
        ],
        out_specs=pl.BlockSpec((_MM_BLK, D_PAD), lambda i: (i, 0)),
        out_shape=jax.ShapeDtypeStruct((NP, D_PAD), jnp.float32),
    )(x, w_pad)


def _ep_body(p_ref, b_ref, o_ref):
    o_ref[...] = p_ref[0] + p_ref[1] + b_ref[0]


_EP_BLK = 2000


def _epilogue(partials, b):
    d_out = b.shape[0]
    bp = jnp.zeros((1, D_PAD), jnp.float32).at[0, :d_out].set(b)
    full = pl.pallas_call(
        _ep_body,
        grid=(N // _EP_BLK,),
        in_specs=[
            pl.BlockSpec((2, _EP_BLK, D_PAD), lambda i: (0, i, 0)),
            pl.BlockSpec((1, D_PAD), lambda i: (0, 0)),
        ],
        out_specs=pl.BlockSpec((_EP_BLK, D_PAD), lambda i: (i, 0)),
        out_shape=jax.ShapeDtypeStruct((N, D_PAD), jnp.float32),
    )(partials, bp)
    return full[:, :d_out]


def _sc_body(h_hbm, e_hbm, w_hbm, out_hbm,
             b0, b1, b2, b3, src_v, dst_v, w_v, h_sh, acc_sh,
             gs0, gs1, gs2, gs3, ss0, ss1, ss2, ss3, base, extra):
    c = lax.axis_index("c")
    s = lax.axis_index("s")
    wid = c * NS + s
    bufs = (b0, b1, b2, b3)
    gsems = (gs0, gs1, gs2, gs3)
    ssems = (ss0, ss1, ss2, ss3)

    zeros16 = jnp.zeros((16,), jnp.float32)

    start_blk = wid * base + jnp.minimum(wid, extra)
    pltpu.sync_copy(e_hbm.at[0, pl.ds(start_blk, base)],
                    src_v.at[pl.ds(0, base)])
    pltpu.sync_copy(e_hbm.at[1, pl.ds(start_blk, base)],
                    dst_v.at[pl.ds(0, base)])
    pltpu.sync_copy(w_hbm.at[pl.ds(start_blk, base)],
                    w_v.at[pl.ds(0, base)])
    if extra:
        @pl.when(wid < extra)
        def _():
            eb = start_blk + base
            pltpu.sync_copy(e_hbm.at[0, pl.ds(eb, 1)],
                            src_v.at[pl.ds(base, 1)])
            pltpu.sync_copy(e_hbm.at[1, pl.ds(eb, 1)],
                            dst_v.at[pl.ds(base, 1)])
            pltpu.sync_copy(w_hbm.at[pl.ds(eb, 1)],
                            w_v.at[pl.ds(base, 1)])

    def _z(i, _):
        b0[i, :] = zeros16
        return 0
    lax.fori_loop(0, BLK, _z, 0)

    rows = NP // NS
    rbase = s * rows

    def _zacc(k, _):
        pltpu.sync_copy(b0, acc_sh.at[pl.ds(rbase + k * BLK, BLK)])
        return 0
    lax.fori_loop(0, rows // BLK, _zacc, 0)
    pltpu.sync_copy(h_hbm.at[pl.ds(rbase, rows)], h_sh.at[pl.ds(rbase, rows)])
    plsc.subcore_barrier()

    def start_gather(jb, buf, sem):
        pltpu.async_copy(h_sh.at[src_v.at[jb]], buf, sem)

    def wait_gather(jb, buf, sem):
        pltpu.make_async_copy(h_sh.at[src_v.at[jb]], buf, sem).wait()

    def start_scatter(jb, buf, sem):
        pltpu.async_copy(buf, acc_sh.at[dst_v.at[jb]], sem, add=True)

    def wait_scatter(jb, buf, sem):
        pltpu.make_async_copy(buf, acc_sh.at[dst_v.at[jb]], sem).wait()

    def multiply(buf, jb):
        for g in range(BLK // 16):
            wvec = w_v[jb, pl.ds(g * 16, 16)]
            for k in range(16):
                wb = jnp.take_along_axis(
                    wvec, jnp.full((16,), k, jnp.int32), axis=0)
                o = g * 16 + k
                buf[o, :] = buf[o, :] * wb

    def process(jb, r):
        s2 = (r + 2) % 4
        wait_gather(jb, bufs[r], gsems[r])
        multiply(bufs[r], jb)
        start_scatter(jb, bufs[r], ssems[r])

        @pl.when(jb + 2 < base)
        def _():
            @pl.when(jb - 2 >= 0)
            def _():
                wait_scatter(jb - 2, bufs[s2], ssems[s2])
            start_gather(jb + 2, bufs[s2], gsems[s2])

    start_gather(0, b0, gs0)
    start_gather(1, b1, gs1)

    def _quad(j4, _):
        for r in range(4):
            process(j4 * 4 + r, r)
        return 0
    nq = base // 4
    lax.fori_loop(0, nq, _quad, 0)
    for r in range(base % 4):
        process(jnp.int32(nq * 4 + r), r)
    for t in range(min(4, base)):
        blk = base - min(4, base) + t
        wait_scatter(jnp.int32(blk), bufs[blk % 4], ssems[blk % 4])

    if extra:
        @pl.when(wid < extra)
        def _():
            jb = jnp.int32(base)
            start_gather(jb, b0, gs0)
            wait_gather(jb, b0, gs0)
            multiply(b0, jb)
            start_scatter(jb, b0, ss0)
            wait_scatter(jb, b0, ss0)

    plsc.subcore_barrier()
    pltpu.sync_copy(acc_sh.at[pl.ds(rbase, rows)],
                    out_hbm.at[c, pl.ds(rbase, rows)])


def _edge_aggregate(h, edges, w, base, extra):
    mesh = plsc.VectorSubcoreMesh(core_axis_name="c", subcore_axis_name="s")
    body = functools.partial(_sc_body, base=base, extra=extra)
    nslab = base + (1 if extra else 0)
    return pl.kernel(
        body,
        out_type=jax.ShapeDtypeStruct((NC, NP, D_PAD), jnp.float32),
        mesh=mesh,
        compiler_params=pltpu.CompilerParams(use_tc_tiling_on_sc=False),
        scratch_types=[
            pltpu.VMEM((BLK, D_PAD), jnp.float32),
            pltpu.VMEM((BLK, D_PAD), jnp.float32),
            pltpu.VMEM((BLK, D_PAD), jnp.float32),
            pltpu.VMEM((BLK, D_PAD), jnp.float32),
            pltpu.VMEM((nslab, BLK), jnp.int32),
            pltpu.VMEM((nslab, BLK), jnp.int32),
            pltpu.VMEM((nslab, BLK), jnp.float32),
            pltpu.VMEM_SHARED((NP, D_PAD), jnp.float32),
            pltpu.VMEM_SHARED((NP, D_PAD), jnp.float32),
            pltpu.SemaphoreType.DMA,
            pltpu.SemaphoreType.DMA,
            pltpu.SemaphoreType.DMA,
            pltpu.SemaphoreType.DMA,
            pltpu.SemaphoreType.DMA,
            pltpu.SemaphoreType.DMA,
            pltpu.SemaphoreType.DMA,
            pltpu.SemaphoreType.DMA,
        ],
    )(h, edges, w)


@jax.jit
def kernel(x, edge_index, w_mul, W, b):
    e = edge_index.shape[1]
    edges = edge_index.astype(jnp.int32)
    w = w_mul
    if e % BLK:
        pad = BLK - e % BLK
        edges = jnp.concatenate(
            [edges, jnp.zeros((2, pad), jnp.int32)], axis=1)
        w = jnp.concatenate([w, jnp.zeros((pad,), jnp.float32)])
        e += pad
    nblk = e // BLK
    base, extra = divmod(nblk, NW)
    edges = edges.reshape(2, nblk, BLK)
    w = w.reshape(nblk, BLK)

    w_pad = jnp.zeros((D_IN, D_PAD), jnp.float32).at[:, : W.shape[1]].set(W)
    h = _matmul(x, w_pad)

    partials = _edge_aggregate(h, edges, w, base, extra)
    return _epilogue(partials, b)
